# baseline (speedup 1.0000x reference)
import functools

import jax
import jax.numpy as jnp
from jax import lax
from jax.experimental import pallas as pl
from jax.experimental.pallas import tpu as pltpu
from jax.experimental.pallas import tpu_sc as plsc

M_ROWS = 100000
D = 256
B_ROWS = 16384
NC, NS = 2, 16
NW = NC * NS
BPW = B_ROWS // NW
CH = 128
NCH = BPW // CH

_mesh = plsc.VectorSubcoreMesh(core_axis_name="c", subcore_axis_name="s")


def _wid():
  return lax.axis_index("s") * NC + lax.axis_index("c")


def _gather_pipeline(src_hbm, idx_v, out_hbm, base, rows_v, sg, sw):
  gathers = [None] * NCH
  writes = [None] * NCH
  gathers[0] = pltpu.async_copy(src_hbm.at[idx_v.at[0]], rows_v.at[0], sg[0])
  for ch in range(NCH):
    b = ch % 2
    if ch + 1 < NCH:
      nb = (ch + 1) % 2
      if writes[ch - 1] is not None:
        writes[ch - 1].wait()
      gathers[ch + 1] = pltpu.async_copy(
          src_hbm.at[idx_v.at[ch + 1]], rows_v.at[nb], sg[nb])
    gathers[ch].wait()
    writes[ch] = pltpu.async_copy(
        rows_v.at[b], out_hbm.at[pl.ds(base + ch * CH, CH)], sw[b])
  writes[NCH - 2].wait()
  writes[NCH - 1].wait()


@functools.partial(
    pl.kernel,
    mesh=_mesh,
    out_type=jax.ShapeDtypeStruct((B_ROWS, D), jnp.float32),
    scratch_types=[
        pltpu.VMEM((NCH, CH), jnp.int32),
        pltpu.VMEM((NCH, CH), jnp.float32),
        pltpu.VMEM((2, CH, D), jnp.float32),
        pltpu.SemaphoreType.DMA,
        pltpu.SemaphoreType.DMA,
        pltpu.SemaphoreType.DMA,
        pltpu.SemaphoreType.DMA,
    ],
)
def _sc_gather_h(table_hbm, ids_hbm, tsw_hbm, lu_ref, out_hbm,
                 idx_v, ts_v, rows_v, sg0, sg1, sw0, sw1):
  wid = _wid()
  pltpu.sync_copy(ids_hbm.at[wid], idx_v)
  pltpu.sync_copy(tsw_hbm.at[wid], ts_v)
  for ch in range(NCH):
    pltpu.sync_copy(ts_v.at[ch], lu_ref.at[idx_v.at[ch]])
  _gather_pipeline(table_hbm, idx_v, out_hbm, wid * BPW, rows_v,
                   (sg0, sg1), (sw0, sw1))


@functools.partial(
    pl.kernel,
    mesh=_mesh,
    out_type=jax.ShapeDtypeStruct((B_ROWS, D), jnp.float32),
    scratch_types=[
        pltpu.VMEM((NCH, CH), jnp.int32),
        pltpu.VMEM((2, CH, D), jnp.float32),
        pltpu.SemaphoreType.DMA,
        pltpu.SemaphoreType.DMA,
        pltpu.SemaphoreType.DMA,
        pltpu.SemaphoreType.DMA,
    ],
)
def _sc_gather_msgs(msgs_hbm, occ_hbm, out_hbm, idx_v, rows_v, sg0, sg1, sw0, sw1):
  wid = _wid()
  pltpu.sync_copy(occ_hbm.at[wid], idx_v)
  _gather_pipeline(msgs_hbm, idx_v, out_hbm, wid * BPW, rows_v,
                   (sg0, sg1), (sw0, sw1))


BLK = 512


def _gru_block(x_ref, h_ref, wih_ref, whh_ref, bih_ref, bhh_ref,
               fcw_ref, fcb_ref, linw_ref, linb_ref, out_ref):
  x = x_ref[...]
  h = h_ref[...]
  gi = jnp.dot(x, wih_ref[...], preferred_element_type=jnp.float32) + bih_ref[...]
  gh = jnp.dot(h, whh_ref[...], preferred_element_type=jnp.float32) + bhh_ref[...]
  r = jax.nn.sigmoid(gi[:, 0:D] + gh[:, 0:D])
  z = jax.nn.sigmoid(gi[:, D:2 * D] + gh[:, D:2 * D])
  n = jnp.tanh(gi[:, 2 * D:3 * D] + r * gh[:, 2 * D:3 * D])
  hy = (1.0 - z) * n + z * h
  pred = jnp.dot(hy, fcw_ref[...], preferred_element_type=jnp.float32) + fcb_ref[...]
  out_ref[...] = jnp.dot(pred, linw_ref[...], preferred_element_type=jnp.float32) + linb_ref[...]


_gru = pl.pallas_call(
    _gru_block,
    grid=(B_ROWS // BLK,),
    in_specs=[
        pl.BlockSpec((BLK, D), lambda i: (i, 0)),
        pl.BlockSpec((BLK, D), lambda i: (i, 0)),
        pl.BlockSpec((D, 3 * D), lambda i: (0, 0)),
        pl.BlockSpec((D, 3 * D), lambda i: (0, 0)),
        pl.BlockSpec((1, 3 * D), lambda i: (0, 0)),
        pl.BlockSpec((1, 3 * D), lambda i: (0, 0)),
        pl.BlockSpec((D, 64), lambda i: (0, 0)),
        pl.BlockSpec((1, 64), lambda i: (0, 0)),
        pl.BlockSpec((64, D), lambda i: (0, 0)),
        pl.BlockSpec((1, D), lambda i: (0, 0)),
    ],
    out_specs=pl.BlockSpec((BLK, D), lambda i: (i, 0)),
    out_shape=jax.ShapeDtypeStruct((B_ROWS, D), jnp.float32),
)


@functools.partial(
    pl.kernel,
    mesh=_mesh,
    out_type=(),
    scratch_types=[
        pltpu.VMEM((NCH, CH), jnp.int32),
        pltpu.VMEM((2, CH, D), jnp.float32),
        pltpu.SemaphoreType.DMA,
        pltpu.SemaphoreType.DMA,
        pltpu.SemaphoreType.DMA,
        pltpu.SemaphoreType.DMA,
    ],
)
def _sc_scatter(newmem_hbm, ids_hbm, table_ref, ids_v, rows_v, sl0, sl1, ss0, ss1):
  wid = _wid()
  base = wid * BPW
  sl = (sl0, sl1)
  ss = (ss0, ss1)
  pltpu.sync_copy(ids_hbm.at[wid], ids_v)
  loads = [None] * NCH
  scatters = [None] * NCH
  loads[0] = pltpu.async_copy(newmem_hbm.at[pl.ds(base, CH)], rows_v.at[0], sl[0])
  for ch in range(NCH):
    b = ch % 2
    if ch + 1 < NCH:
      nb = (ch + 1) % 2
      if scatters[ch - 1] is not None:
        scatters[ch - 1].wait()
      loads[ch + 1] = pltpu.async_copy(
          newmem_hbm.at[pl.ds(base + (ch + 1) * CH, CH)], rows_v.at[nb], sl[nb])
    loads[ch].wait()
    scatters[ch] = pltpu.async_copy(rows_v.at[b], table_ref.at[ids_v.at[ch]], ss[b])
  scatters[NCH - 2].wait()
  scatters[NCH - 1].wait()


def kernel(memory_table, last_update, unique_node_ids, unique_messages,
           timestamps, w_ih, w_hh, b_ih, b_hh, fc_w, fc_b, lin_w, lin_b):
  ids = unique_node_ids

  table_ref = jax.new_ref(memory_table)
  lu_ref = jax.new_ref(last_update)

  iota = jnp.arange(B_ROWS, dtype=jnp.int32)
  key = jnp.sort((ids << 14) | iota)
  sid = key >> 14
  sp = key & (B_ROWS - 1)
  boundary = jnp.concatenate([sid[1:] != sid[:-1],
                              jnp.ones((1,), jnp.bool_)])
  b_idx = jnp.where(boundary, iota, B_ROWS)
  nb = lax.cummin(b_idx, axis=0, reverse=True)
  winner_sorted = sp[nb]
  lastocc = jnp.zeros((B_ROWS,), jnp.int32).at[sp].set(
      winner_sorted, unique_indices=True)
  tsw = jnp.take(timestamps, lastocc)

  ids3 = ids.reshape(NW, NCH, CH)
  h = _sc_gather_h(memory_table, ids3, tsw.reshape(NW, NCH, CH), lu_ref)
  msgs_w = _sc_gather_msgs(unique_messages, lastocc.reshape(NW, NCH, CH))

  new_mem = _gru(
      msgs_w, h,
      w_ih.T, w_hh.T,
      b_ih.reshape(1, 3 * D), b_hh.reshape(1, 3 * D),
      fc_w.T, fc_b.reshape(1, 64),
      lin_w.T, lin_b.reshape(1, D),
  )

  _sc_scatter(new_mem, ids3, table_ref)
  return jax.freeze(table_ref), jax.freeze(lu_ref)
